# Initial kernel scaffold; baseline (speedup 1.0000x reference)
import jax
import jax.numpy as jnp
from jax.experimental import pallas as pl


def kernel(lbl, W):
    raise NotImplementedError("write your pallas kernel here")


# speedup vs baseline: 3123.5357x; 3123.5357x over previous
import dataclasses

import jax
import jax.numpy as jnp
from jax import lax
from jax.experimental import pallas as pl
from jax.experimental.pallas import tpu as pltpu
from jax.experimental.pallas import tpu_sc as plsc

HOP = 128
FFT = 4096
NUM_LABELS = 128
SEQ = 32768
PAD = FFT // 2
PADDED = SEQ + 2 * PAD
OUT_T = (PADDED - FFT) // HOP + 1
BATCH = 8
WORKERS_PER_BATCH = 4
NUM_WORKERS = BATCH * WORKERS_PER_BATCH
WIN_PER_WORKER = 65
SPAN = (WIN_PER_WORKER - 1) * HOP + FFT
RES_PAD = 80
LANES = 16


def _sc_compiler_params():
    cp = pltpu.CompilerParams()
    if "needs_layout_passes" in pltpu.CompilerParams.__dataclass_fields__:
        cp = dataclasses.replace(cp, needs_layout_passes=False)
    return cp


def _label_argmax_sc(padded):
    mesh = plsc.VectorSubcoreMesh(core_axis_name="c", subcore_axis_name="s")

    def run(padded):
        @pl.kernel(
            out_type=jax.ShapeDtypeStruct((NUM_WORKERS, RES_PAD), jnp.int32),
            mesh=mesh,
            scratch_types=[
                pltpu.VMEM((SPAN,), jnp.int32),
                pltpu.VMEM((NUM_LABELS,), jnp.int32),
                pltpu.VMEM((RES_PAD,), jnp.int32),
            ],
            compiler_params=_sc_compiler_params(),
        )
        def k(lbl_hbm, out_hbm, lbl_v, hist_v, res_v):
            wid = lax.axis_index("c") * 16 + lax.axis_index("s")
            b = wid // WORKERS_PER_BATCH
            q = wid % WORKERS_PER_BATCH

            pltpu.sync_copy(lbl_hbm.at[b, pl.ds(q * 8192, SPAN)], lbl_v)

            lane_iota = lax.iota(jnp.int32, LANES)

            zeros16 = jnp.zeros((LANES,), jnp.int32)
            for j in range(NUM_LABELS // LANES):
                hist_v[pl.ds(j * LANES, LANES)] = zeros16

            def hist_add(off, sign):
                v = lbl_v[pl.ds(off, LANES)]
                cnt, last = plsc.scan_count(v)
                plsc.addupdate_scatter(
                    hist_v, [v], cnt if sign > 0 else -cnt, mask=last)

            @pl.loop(0, FFT // LANES)
            def _(g):
                hist_add(g * LANES, 1)

            @pl.loop(0, WIN_PER_WORKER)
            def _(i):
                @pl.when(i > 0)
                def _():
                    base = (i - 1) * HOP
                    for j in range(HOP // LANES):
                        hist_add(base + j * LANES, -1)
                        hist_add(base + FFT + j * LANES, 1)

                m = jnp.full((LANES,), -1, jnp.int32)
                for j in range(NUM_LABELS // LANES):
                    h = hist_v[pl.ds(j * LANES, LANES)]
                    key = h * NUM_LABELS + (NUM_LABELS - 1 - j * LANES) - lane_iota
                    m = jnp.maximum(m, key)
                smax = jnp.max(m)
                best = (NUM_LABELS - 1) - (smax & (NUM_LABELS - 1))

                grp = i & -LANES
                lane = i & (LANES - 1)
                cur = res_v[pl.ds(grp, LANES)]
                res_v[pl.ds(grp, LANES)] = jnp.where(
                    lane_iota == lane, best, cur)

            pltpu.sync_copy(res_v, out_hbm.at[wid])

        return k(padded)

    return run(padded)


def kernel(lbl, W):
    del W
    left = lbl[:, 1:PAD + 1][:, ::-1]
    right = lbl[:, SEQ - 1 - PAD:SEQ - 1][:, ::-1]
    padded = jnp.concatenate([left, lbl, right], axis=1)

    res = _label_argmax_sc(padded)
    res = res.reshape(BATCH, WORKERS_PER_BATCH, RES_PAD)
    out = jnp.concatenate(
        [res[:, 0, :64], res[:, 1, :64], res[:, 2, :64], res[:, 3, :65]],
        axis=1)
    return out


# speedup vs baseline: 4962.3746x; 1.5887x over previous
import dataclasses

import jax
import jax.numpy as jnp
from jax import lax
from jax.experimental import pallas as pl
from jax.experimental.pallas import tpu as pltpu
from jax.experimental.pallas import tpu_sc as plsc

HOP = 128
FFT = 4096
NUM_LABELS = 128
SEQ = 32768
PAD = FFT // 2
PADDED = SEQ + 2 * PAD
OUT_T = (PADDED - FFT) // HOP + 1
BATCH = 8
WORKERS_PER_BATCH = 4
NUM_WORKERS = BATCH * WORKERS_PER_BATCH
WIN_PER_WORKER = 65
SPAN = (WIN_PER_WORKER - 1) * HOP + FFT
RES_PAD = 80
LANES = 16
GROUPS = HOP // LANES


def _sc_compiler_params():
    cp = pltpu.CompilerParams()
    if "needs_layout_passes" in pltpu.CompilerParams.__dataclass_fields__:
        cp = dataclasses.replace(cp, needs_layout_passes=False)
    return cp


def _label_argmax_sc(padded):
    mesh = plsc.VectorSubcoreMesh(core_axis_name="c", subcore_axis_name="s")

    @pl.kernel(
        out_type=jax.ShapeDtypeStruct((NUM_WORKERS, RES_PAD), jnp.int32),
        mesh=mesh,
        scratch_types=[
            pltpu.VMEM((SPAN,), jnp.int32),
            pltpu.VMEM((NUM_LABELS,), jnp.int32),
            pltpu.VMEM((RES_PAD,), jnp.int32),
        ],
        compiler_params=_sc_compiler_params(),
    )
    def k(lbl_hbm, out_hbm, lbl_v, hist_v, res_v):
        wid = lax.axis_index("c") * 16 + lax.axis_index("s")
        b = wid // WORKERS_PER_BATCH
        q = wid % WORKERS_PER_BATCH

        pltpu.sync_copy(lbl_hbm.at[b, pl.ds(q * 8192, SPAN)], lbl_v)

        lane_iota = lax.iota(jnp.int32, LANES)
        top_lane = lane_iota == (LANES - 1)
        key_offs = [
            jnp.full((LANES,), NUM_LABELS - 1 - j * LANES, jnp.int32) - lane_iota
            for j in range(GROUPS)
        ]

        zeros16 = jnp.zeros((LANES,), jnp.int32)
        for j in range(NUM_LABELS // LANES):
            hist_v[pl.ds(j * LANES, LANES)] = zeros16

        def hist_update(offs_signs):
            vs = [lbl_v[pl.ds(off, LANES)] for off, _ in offs_signs]
            scans = [plsc.scan_count(v) for v in vs]
            for (off, sign), v, (cnt, last) in zip(offs_signs, vs, scans):
                plsc.addupdate_scatter(
                    hist_v, [v], cnt if sign > 0 else -cnt, mask=last)

        def argmax_store(slot):
            m = hist_v[pl.ds(0, LANES)] * NUM_LABELS + key_offs[0]
            for j in range(1, GROUPS):
                key = hist_v[pl.ds(j * LANES, LANES)] * NUM_LABELS + key_offs[j]
                m = jnp.maximum(m, key)
            s = lax.sort(m)
            best = (NUM_LABELS - 1) - (s & (NUM_LABELS - 1))
            idx = jnp.full((LANES,), slot, jnp.int32)
            plsc.store_scatter(res_v, [idx], best, mask=top_lane)

        @pl.loop(0, FFT // HOP)
        def _(c):
            hist_update([(c * HOP + j * LANES, 1) for j in range(GROUPS)])

        argmax_store(0)

        @pl.loop(0, WIN_PER_WORKER - 1)
        def _(i):
            base = i * HOP
            hist_update(
                [(base + j * LANES, -1) for j in range(GROUPS)]
                + [(base + FFT + j * LANES, 1) for j in range(GROUPS)])
            argmax_store(i + 1)

        pltpu.sync_copy(res_v, out_hbm.at[wid])

    return k(padded)


def kernel(lbl, W):
    del W
    rev = lbl[:, ::-1]
    padded = jnp.concatenate(
        [rev[:, SEQ - 1 - PAD:SEQ - 1], lbl, rev[:, 1:PAD + 1]], axis=1)

    res = _label_argmax_sc(padded)
    res = res.reshape(BATCH, WORKERS_PER_BATCH, RES_PAD)
    out = jnp.concatenate(
        [res[:, 0, :64], res[:, 1, :64], res[:, 2, :64], res[:, 3, :65]],
        axis=1)
    return out
